# Initial kernel scaffold; baseline (speedup 1.0000x reference)
import jax
import jax.numpy as jnp
from jax.experimental import pallas as pl


def kernel(x, edge_index, mask_x_position, emb, W1, b1, W2, b2):
    raise NotImplementedError("write your pallas kernel here")


# speedup vs baseline: 12.3701x; 12.3701x over previous
import functools

import jax
import jax.numpy as jnp
from jax import lax
from jax.experimental import pallas as pl
from jax.experimental.pallas import tpu as pltpu
from jax.experimental.pallas import tpu_sc as plsc

NC, NS = 2, 16
NW = NC * NS
N = 10000
NP = 10240
E = 320000
CH = 128
NCHUNK = (E + NW * CH - 1) // (NW * CH)
EPW = NCHUNK * CH
EP = EPW * NW
D = 128
V = 10000
VP = 10240
M = 1000
MP = 1024
RPW = NP // NW
RSL = NP // NS

_mesh = plsc.VectorSubcoreMesh(core_axis_name="c", subcore_axis_name="s")
_sc_params = pltpu.CompilerParams(needs_layout_passes=False)


def _worker_ids():
    c = lax.axis_index("c")
    s = lax.axis_index("s")
    return c, s, c * NS + s


def _sc_prep_body(xp, dstp, emb, z8, ones8, h_out, deg_out,
                  xv, rows64, dstv, ones_v, deg_s):
    c, s, w = _worker_ids()
    for k in range(RPW // 64):
        base = w * RPW + k * 64
        pltpu.sync_copy(xp.at[pl.ds(base, 64)], xv)
        pltpu.sync_copy(emb.at[xv], rows64)
        pltpu.sync_copy(rows64, h_out.at[pl.ds(base, 64)])
    pltpu.sync_copy(z8.at[pl.ds(s * RSL, RSL)], deg_s.at[pl.ds(s * RSL, RSL)])
    pltpu.sync_copy(ones8, ones_v)
    pltpu.sync_copy(dstp.at[w], dstv)
    plsc.subcore_barrier()

    def body(j, carry):
        pltpu.sync_copy(ones_v, deg_s.at[dstv.at[j]], add=True)
        return carry

    lax.fori_loop(0, NCHUNK, body, 0)
    plsc.subcore_barrier()
    pltpu.sync_copy(deg_s.at[pl.ds(s * RSL, RSL)],
                    deg_out.at[c, pl.ds(s * RSL, RSL)])


_sc_prep = pl.kernel(
    _sc_prep_body,
    out_type=(
        jax.ShapeDtypeStruct((NP, D), jnp.float32),
        jax.ShapeDtypeStruct((NC, NP, 8), jnp.float32),
    ),
    mesh=_mesh,
    scratch_types=[
        pltpu.VMEM((64,), jnp.int32),
        pltpu.VMEM((64, D), jnp.float32),
        pltpu.VMEM((NCHUNK, CH), jnp.int32),
        pltpu.VMEM((CH, 8), jnp.float32),
        pltpu.VMEM_SHARED((NP, 8), jnp.float32),
    ],
)


def _sc_agg_body(masked, m, srcp, dstp, z128, maskp, dinvb, *refs):
    if masked:
        pm_out, dm_out, srcv, dstv, rows, agg_s, mv, rows64 = refs
    else:
        p_out, srcv, dstv, rows, agg_s = refs
    c, s, w = _worker_ids()
    sl = pl.ds(s * RSL, RSL)

    @pl.when(c == 0)
    def _():
        pltpu.sync_copy(m.at[sl], agg_s.at[sl])

    @pl.when(c != 0)
    def _():
        pltpu.sync_copy(z128.at[sl], agg_s.at[sl])

    pltpu.sync_copy(srcp.at[w], srcv)
    pltpu.sync_copy(dstp.at[w], dstv)
    plsc.subcore_barrier()

    def body(j, carry):
        pltpu.sync_copy(m.at[srcv.at[j]], rows)
        pltpu.sync_copy(rows, agg_s.at[dstv.at[j]], add=True)
        return carry

    lax.fori_loop(0, NCHUNK, body, 0)
    plsc.subcore_barrier()
    if not masked:
        pltpu.sync_copy(agg_s.at[sl], p_out.at[c, sl])
    else:
        msl = pl.ds(s * (MP // NS), MP // NS)
        pltpu.sync_copy(maskp.at[msl], mv)
        pltpu.sync_copy(agg_s.at[mv], rows64)
        pltpu.sync_copy(rows64, pm_out.at[c, msl])

        @pl.when(c == 0)
        def _():
            pltpu.sync_copy(dinvb.at[mv], rows64)
            pltpu.sync_copy(rows64, dm_out.at[msl])


_agg_scratch = [
    pltpu.VMEM((NCHUNK, CH), jnp.int32),
    pltpu.VMEM((NCHUNK, CH), jnp.int32),
    pltpu.VMEM((CH, D), jnp.float32),
    pltpu.VMEM_SHARED((NP, D), jnp.float32),
]

_sc_agg_full = pl.kernel(
    functools.partial(_sc_agg_body, False),
    out_type=jax.ShapeDtypeStruct((NC, NP, D), jnp.float32),
    mesh=_mesh,
    scratch_types=list(_agg_scratch),
)

_sc_agg_masked = pl.kernel(
    functools.partial(_sc_agg_body, True),
    out_type=(
        jax.ShapeDtypeStruct((NC, MP, D), jnp.float32),
        jax.ShapeDtypeStruct((MP, D), jnp.float32),
    ),
    mesh=_mesh,
    compiler_params=_sc_params,
    scratch_types=list(_agg_scratch) + [
        pltpu.VMEM((MP // NS,), jnp.int32),
        pltpu.VMEM((MP // NS, D), jnp.float32),
    ],
)


def _tc_scale_body(h_ref, deg_ref, m_ref, dinv8_ref, dinvb_ref):
    deg = deg_ref[0] + deg_ref[1] + 1.0
    dinv = lax.rsqrt(deg)
    dinv8_ref[...] = dinv
    dinvb_ref[...] = jnp.broadcast_to(dinv[:, 0:1], (128, D))
    m_ref[...] = h_ref[...] * dinv[:, 0:1]


def _tc_scale(h, degp):
    return pl.pallas_call(
        _tc_scale_body,
        grid=(NP // 128,),
        in_specs=[
            pl.BlockSpec((128, D), lambda i: (i, 0)),
            pl.BlockSpec((NC, 128, 8), lambda i: (0, i, 0)),
        ],
        out_specs=[
            pl.BlockSpec((128, D), lambda i: (i, 0)),
            pl.BlockSpec((128, 8), lambda i: (i, 0)),
            pl.BlockSpec((128, D), lambda i: (i, 0)),
        ],
        out_shape=[
            jax.ShapeDtypeStruct((NP, D), jnp.float32),
            jax.ShapeDtypeStruct((NP, 8), jnp.float32),
            jax.ShapeDtypeStruct((NP, D), jnp.float32),
        ],
    )(h, degp)


def _tc_mid_body(p_ref, dinv8_ref, w_ref, b_ref, m2_ref):
    col = dinv8_ref[:, 0:1]
    agg = (p_ref[0] + p_ref[1]) * col
    z = jnp.dot(agg, w_ref[...], preferred_element_type=jnp.float32)
    m2_ref[...] = jnp.maximum(z + b_ref[...][None, :], 0.0) * col


def _tc_mid(p, dinv8, W1, b1):
    return pl.pallas_call(
        _tc_mid_body,
        grid=(NP // 128,),
        in_specs=[
            pl.BlockSpec((NC, 128, D), lambda i: (0, i, 0)),
            pl.BlockSpec((128, 8), lambda i: (i, 0)),
            pl.BlockSpec((D, D), lambda i: (0, 0)),
            pl.BlockSpec((D,), lambda i: (0,)),
        ],
        out_specs=pl.BlockSpec((128, D), lambda i: (i, 0)),
        out_shape=jax.ShapeDtypeStruct((NP, D), jnp.float32),
    )(p, dinv8, W1, b1)


def _tc_head_body(pm_ref, dm_ref, w_ref, b_ref, out_ref):
    col = dm_ref[:, 0:1]
    aggm = (pm_ref[0] + pm_ref[1]) * col
    logits = jnp.dot(aggm, w_ref[...], preferred_element_type=jnp.float32)
    logits = logits + b_ref[...][None, :]
    mx = jnp.max(logits, axis=1, keepdims=True)
    lse = jnp.log(jnp.sum(jnp.exp(logits - mx), axis=1, keepdims=True))
    out_ref[...] = logits - mx - lse


def _tc_head(pm, dm, W2p, b2p):
    return pl.pallas_call(
        _tc_head_body,
        grid=(MP // 128,),
        in_specs=[
            pl.BlockSpec((NC, 128, D), lambda i: (0, i, 0)),
            pl.BlockSpec((128, D), lambda i: (i, 0)),
            pl.BlockSpec((D, VP), lambda i: (0, 0)),
            pl.BlockSpec((VP,), lambda i: (0,)),
        ],
        out_specs=pl.BlockSpec((128, VP), lambda i: (i, 0)),
        out_shape=jax.ShapeDtypeStruct((MP, VP), jnp.float32),
    )(pm, dm, W2p, b2p)


def kernel(x, edge_index, mask_x_position, emb, W1, b1, W2, b2):
    xp = jnp.pad(x[:, 0].astype(jnp.int32), (0, NP - N))
    src = jnp.pad(edge_index[0].astype(jnp.int32), (0, EP - E))
    dst = jnp.pad(edge_index[1].astype(jnp.int32), (0, EP - E),
                  constant_values=NP - 1)
    srcp = src.reshape(NW, NCHUNK, CH)
    dstp = dst.reshape(NW, NCHUNK, CH)
    maskp = jnp.pad(mask_x_position.astype(jnp.int32), (0, MP - M))
    z8 = jnp.zeros((NP, 8), jnp.float32)
    z128 = jnp.zeros((NP, D), jnp.float32)
    ones8 = jnp.ones((CH, 8), jnp.float32)
    W2p = jnp.pad(W2, ((0, 0), (0, VP - V)))
    b2p = jnp.pad(b2, (0, VP - V), constant_values=-1e30)

    h, degp = _sc_prep(xp, dstp, emb, z8, ones8)
    m1, dinv8, dinvb = _tc_scale(h, degp)
    p1 = _sc_agg_full(m1, srcp, dstp, z128, maskp, dinvb)
    m2 = _tc_mid(p1, dinv8, W1, b1)
    pm, dm = _sc_agg_masked(m2, srcp, dstp, z128, maskp, dinvb)
    outp = _tc_head(pm, dm, W2p, b2p)
    return outp[:M, :V]
